# baseline (speedup 1.0000x reference)
import functools

import jax
import jax.numpy as jnp
from jax.experimental import pallas as pl
from jax.experimental.pallas import tpu as pltpu

_PATCH = 5
_FEAT = 75
_LANES = 128


def _unfold_t(x, k):
    C, H, W = x.shape[1], x.shape[2], x.shape[3]
    oh, ow = H - k + 1, W - k + 1
    pats = jnp.stack(
        [x[0, :, i:i + oh, j:j + ow] for i in range(k) for j in range(k)],
        axis=1,
    )
    return pats.reshape(C * k * k, oh * ow)


def _nn_body(n_total, xn_ref, y_ref, x2_ref, y2_ref, loss_ref, idx_ref,
             loss_acc):
    i = pl.program_id(0)
    ni = pl.num_programs(0)

    nb = idx_ref.shape[0]
    xn = xn_ref[0]
    x2 = x2_ref[pl.ds(i * nb, nb), :]
    m_pad = y_ref.shape[1]
    n_chunks = m_pad // _LANES

    lane = jax.lax.broadcasted_iota(
        jnp.int32, (1, _LANES), 1).astype(jnp.float32)

    acc = None
    idx = None
    for k in range(n_chunks):
        yk = y_ref[:, k * _LANES:(k + 1) * _LANES]
        y2k = y2_ref[:, k * _LANES:(k + 1) * _LANES]
        xyk = jax.lax.dot_general(
            xn, yk, (((0,), (0,)), ((), ())),
            preferred_element_type=jnp.float32)
        dk = (x2 + xyk) + y2k
        col = jnp.float32(k * _LANES) + lane
        if acc is None:
            acc = dk
            idx = col + jnp.zeros((nb, _LANES), jnp.float32)
        else:
            upd = dk < acc
            acc = jnp.where(upd, dk, acc)
            idx = jnp.where(upd, col, idx)

    bm = jnp.min(acc, axis=1, keepdims=True)
    cand = jnp.min(
        jnp.where(acc == bm, idx, jnp.float32(3.0e8)),
        axis=1, keepdims=True)
    idx_ref[...] = cand.astype(jnp.int32)

    @pl.when(i == 0)
    def _zero_loss():
        loss_acc[...] = jnp.zeros_like(loss_acc)

    loss_acc[...] += jnp.sum(bm, axis=(0, 1), keepdims=True)

    @pl.when(i == ni - 1)
    def _loss():
        loss_ref[...] = loss_acc[...] / jnp.float32(n_total)


@jax.jit
def kernel(crop, original_image):
    Xt = _unfold_t(crop, _PATCH)
    n = Xt.shape[1]
    k = _PATCH

    img = original_image
    C, H, W = img.shape[1], img.shape[2], img.shape[3]
    oh = H - k + 1
    ipad = jnp.pad(img[0], ((0, 0), (0, 0), (0, k - 1)))
    shifts = [ipad[:, :, j:j + W] for j in range(k)]
    Yt = jnp.stack(
        [shifts[j][c, i:i + oh, :] for c in range(C)
         for i in range(k) for j in range(k)],
        axis=0,
    ).reshape(_FEAT, oh * W)
    m_pad = oh * W

    nb = 360
    x2 = jnp.sum(Xt ** 2, axis=0)[:, None]
    y2 = jnp.sum(Yt ** 2, axis=0, keepdims=True)
    q_of = jax.lax.broadcasted_iota(jnp.int32, (1, m_pad), 1) % W
    y2p = jnp.where(q_of < oh, y2, jnp.inf)
    Xn = (Xt * jnp.float32(-2.0)).reshape(_FEAT, n // nb, nb).swapaxes(0, 1)
    Yp = Yt

    grid = (n // nb,)
    loss2d, idx2d = pl.pallas_call(
        functools.partial(_nn_body, n),
        grid=grid,
        in_specs=[
            pl.BlockSpec((1, _FEAT, nb), lambda i: (i, 0, 0)),
            pl.BlockSpec((_FEAT, m_pad), lambda i: (0, 0)),
            pl.BlockSpec((n, 1), lambda i: (0, 0)),
            pl.BlockSpec((1, m_pad), lambda i: (0, 0)),
        ],
        out_specs=[
            pl.BlockSpec((1, 1), lambda i: (0, 0)),
            pl.BlockSpec((nb, 1), lambda i: (i, 0)),
        ],
        out_shape=[
            jax.ShapeDtypeStruct((1, 1), jnp.float32),
            jax.ShapeDtypeStruct((n, 1), jnp.int32),
        ],
        scratch_shapes=[
            pltpu.VMEM((1, 1), jnp.float32),
        ],
        compiler_params=pltpu.CompilerParams(
            dimension_semantics=("arbitrary",)),
    )(Xn, Yp, x2, y2p)
    idx_grid = idx2d[:, 0]
    idx_ref_order = idx_grid - (k - 1) * (idx_grid // W)
    return loss2d[0, 0], idx_ref_order


# speedup vs baseline: 1.1742x; 1.1742x over previous
import functools

import jax
import jax.numpy as jnp
from jax.experimental import pallas as pl
from jax.experimental.pallas import tpu as pltpu

_PATCH = 5
_FEAT = 75
_LANES = 128
_C = 3
_W = 128
_OH = 124


def _unfold_t(x, k):
    C, H, W = x.shape[1], x.shape[2], x.shape[3]
    oh, ow = H - k + 1, W - k + 1
    pats = jnp.stack(
        [x[0, :, i:i + oh, j:j + ow] for i in range(k) for j in range(k)],
        axis=1,
    )
    return pats.reshape(C * k * k, oh * ow)


def _nn_body(n_total, xn_ref, img_ref, x2_ref, loss_ref, idx_ref,
             yt_s, y2_s, loss_acc):
    i = pl.program_id(0)
    ni = pl.num_programs(0)

    nb = idx_ref.shape[0]
    m_pad = yt_s.shape[1]
    n_chunks = m_pad // _LANES

    lane = jax.lax.broadcasted_iota(
        jnp.int32, (1, _LANES), 1).astype(jnp.float32)

    @pl.when(i == 0)
    def _build_keys():
        lane_i = jax.lax.broadcasted_iota(jnp.int32, (1, _LANES), 1)
        for p in range(n_chunks):
            rows = []
            for c in range(_C):
                for ii in range(_PATCH):
                    r0 = c * _W + p + ii
                    row = img_ref[r0:r0 + 1, :]
                    rows.append(row)
                    for jj in range(1, _PATCH):
                        rows.append(jnp.concatenate(
                            [row[:, jj:], row[:, :jj]], axis=1))
            y2 = rows[0] * rows[0]
            for r in rows[1:]:
                y2 = y2 + r * r
            y2_s[:, p * _LANES:(p + 1) * _LANES] = jnp.where(
                lane_i < _OH, y2, jnp.inf)
            for b in range(_FEAT // 5):
                yt_s[b * 5:(b + 1) * 5, p * _LANES:(p + 1) * _LANES] = (
                    jnp.concatenate(rows[b * 5:(b + 1) * 5], axis=0))

    xn = xn_ref[0]
    x2 = x2_ref[pl.ds(i * nb, nb), :]

    acc = None
    idx = None
    for k in range(n_chunks):
        yk = yt_s[:, k * _LANES:(k + 1) * _LANES]
        y2k = y2_s[:, k * _LANES:(k + 1) * _LANES]
        xyk = jax.lax.dot_general(
            xn, yk, (((0,), (0,)), ((), ())),
            preferred_element_type=jnp.float32)
        dk = (x2 + xyk) + y2k
        col = jnp.float32(k * _LANES) + lane
        if acc is None:
            acc = dk
            idx = col + jnp.zeros((nb, _LANES), jnp.float32)
        else:
            upd = dk < acc
            acc = jnp.where(upd, dk, acc)
            idx = jnp.where(upd, col, idx)

    bm = jnp.min(acc, axis=1, keepdims=True)
    cand = jnp.min(
        jnp.where(acc == bm, idx, jnp.float32(3.0e8)),
        axis=1, keepdims=True)
    idx_ref[...] = cand.astype(jnp.int32)

    @pl.when(i == 0)
    def _zero_loss():
        loss_acc[...] = jnp.zeros_like(loss_acc)

    loss_acc[...] += jnp.sum(bm, axis=(0, 1), keepdims=True)

    @pl.when(i == ni - 1)
    def _loss():
        loss_ref[...] = loss_acc[...] / jnp.float32(n_total)


@jax.jit
def kernel(crop, original_image):
    Xt = _unfold_t(crop, _PATCH)
    n = Xt.shape[1]

    nb = 360
    m_pad = _OH * _W
    img2d = original_image[0].reshape(_C * _W, _W)
    x2 = jnp.sum(Xt ** 2, axis=0)[:, None]
    Xn = (Xt * jnp.float32(-2.0)).reshape(_FEAT, n // nb, nb).swapaxes(0, 1)

    grid = (n // nb,)
    loss2d, idx2d = pl.pallas_call(
        functools.partial(_nn_body, n),
        grid=grid,
        in_specs=[
            pl.BlockSpec((1, _FEAT, nb), lambda i: (i, 0, 0)),
            pl.BlockSpec((_C * _W, _W), lambda i: (0, 0)),
            pl.BlockSpec((n, 1), lambda i: (0, 0)),
        ],
        out_specs=[
            pl.BlockSpec((1, 1), lambda i: (0, 0)),
            pl.BlockSpec((nb, 1), lambda i: (i, 0)),
        ],
        out_shape=[
            jax.ShapeDtypeStruct((1, 1), jnp.float32),
            jax.ShapeDtypeStruct((n, 1), jnp.int32),
        ],
        scratch_shapes=[
            pltpu.VMEM((_FEAT, m_pad), jnp.float32),
            pltpu.VMEM((1, m_pad), jnp.float32),
            pltpu.VMEM((1, 1), jnp.float32),
        ],
        compiler_params=pltpu.CompilerParams(
            dimension_semantics=("arbitrary",)),
    )(Xn, img2d, x2)
    idx_grid = idx2d[:, 0]
    idx_ref_order = idx_grid - (_PATCH - 1) * (idx_grid // _W)
    return loss2d[0, 0], idx_ref_order
